# baseline (speedup 1.0000x reference)
import functools

import jax
import jax.numpy as jnp
from jax import lax
from jax.experimental import pallas as pl
from jax.experimental.pallas import tpu as pltpu
from jax.experimental.pallas import tpu_sc as plsc

B = 4
M = 4096
HW = 4096
KDIM = 256
VDIM = 3
THRESHOLD = 0.35 * 100 / M

TI = 512
NT = M // TI
NW = 32
RPW = B * M // NW
CH = 128
VW = 128
TW = KDIM + VW


def _key(x):
    return 2 * (lax.bitcast_convert_type(x, jnp.int32) - jnp.int32(536870912))


def _rank_body(sc_col_ref, sc_row_ref, mu_col_ref, mu_row_ref,
               r2_ref, ru_ref, wv_ref, pw_ref):
    it = pl.program_id(1)
    si = sc_col_ref[0]
    sj = sc_row_ref[0]
    ki = _key(si)
    kj = _key(sj)
    vi = _key(mu_col_ref[0])
    vj = _key(mu_row_ref[0])
    ii = lax.broadcasted_iota(jnp.int32, (TI, M), 0) + it * TI
    jj = lax.broadcasted_iota(jnp.int32, (TI, M), 1)
    jlt = (jj < ii).astype(jnp.int32)
    ones = jnp.ones((M, 1), jnp.float32)
    dot = functools.partial(lax.dot_general,
                            dimension_numbers=(((1,), (0,)), ((), ())),
                            preferred_element_type=jnp.float32)
    r2 = dot(((kj + jlt) > ki).astype(jnp.float32), ones)
    r2_ref[0] = r2.astype(jnp.int32)
    ru = dot((vj < (vi + jlt)).astype(jnp.float32), ones)
    ru_ref[0] = ru.astype(jnp.int32)
    wv_ref[0] = (si < THRESHOLD).astype(jnp.int32)
    pw = dot(((sj < THRESHOLD) & (jlt == 1)).astype(jnp.float32), ones)
    pw_ref[0] = pw.astype(jnp.int32)


def _ranks(scores, m_u):
    sc_col = scores.reshape(B * NT, TI, 1)
    sc_row = scores.reshape(B, 1, M)
    mu_col = m_u.reshape(B * NT, TI, 1)
    mu_row = m_u.reshape(B, 1, M)
    col_spec = pl.BlockSpec((1, TI, 1), lambda b, i: (b * NT + i, 0, 0))
    row_spec = pl.BlockSpec((1, 1, M), lambda b, i: (b, 0, 0))
    out_spec = pl.BlockSpec((1, TI, 1), lambda b, i: (b * NT + i, 0, 0))
    out_sds = jax.ShapeDtypeStruct((B * NT, TI, 1), jnp.int32)
    r2, ru, wv, pw = pl.pallas_call(
        _rank_body,
        grid=(B, NT),
        in_specs=[col_spec, row_spec, col_spec, row_spec],
        out_specs=[out_spec, out_spec, out_spec, out_spec],
        out_shape=[out_sds, out_sds, out_sds, out_sds],
    )(sc_col, sc_row, mu_col, mu_row)
    return (r2.reshape(B, M), ru.reshape(B, M), wv.reshape(B, M),
            pw.reshape(B, M))


def _compose_body(r2_ref, ru_ref, wv_ref, pw_ref, src_ref):
    b = pl.program_id(0)
    it = pl.program_id(1)
    r2r = r2_ref[0]
    rur = ru_ref[0]
    wvr = wv_ref[0]
    pwr = pw_ref[0]
    jcol = lax.broadcasted_iota(jnp.int32, (TI, M), 0) + it * TI
    ivec = lax.broadcasted_iota(jnp.int32, (M, 1), 0).astype(
        jnp.float32)
    dot = functools.partial(lax.dot_general,
                            dimension_numbers=(((1,), (0,)), ((), ())),
                            preferred_element_type=jnp.float32)
    tmat = (pwr == jcol) & (wvr == 1)
    t = dot(tmat.astype(jnp.float32), ivec).astype(jnp.int32)
    srcw = dot((r2r == t).astype(jnp.float32), ivec).astype(jnp.int32)
    srcd = dot((rur == jcol).astype(jnp.float32), ivec).astype(jnp.int32)
    cnt = jnp.sum(wvr.astype(jnp.float32)).astype(jnp.int32)
    jvec = jcol[:, :1]
    src_ref[0] = jnp.where(jvec < cnt, srcw + b * M,
                           srcd + b * M + B * M)


def _compose(r2, ru, wv, pw):
    rows = [x.reshape(B, 1, M) for x in (r2, ru, wv, pw)]
    row_spec = pl.BlockSpec((1, 1, M), lambda b, i: (b, 0, 0))
    out_spec = pl.BlockSpec((1, TI, 1), lambda b, i: (b * NT + i, 0, 0))
    src = pl.pallas_call(
        _compose_body,
        grid=(B, NT),
        in_specs=[row_spec] * 4,
        out_specs=out_spec,
        out_shape=jax.ShapeDtypeStruct((B * NT, TI, 1), jnp.int32),
    )(*rows)
    return src.reshape(B, M)


_MESH = plsc.VectorSubcoreMesh(core_axis_name="c", subcore_axis_name="s")


@functools.partial(
    pl.kernel,
    out_type=jax.ShapeDtypeStruct((B * M, TW), jnp.float32),
    mesh=_MESH,
    scratch_types=[
        pltpu.VMEM((CH,), jnp.int32),
        pltpu.VMEM((CH, TW), jnp.float32),
        pltpu.SemaphoreType.DMA,
    ],
)
def _sc_gather(tab, srcg, out, idx_v, buf, sem):
    wid = lax.axis_index("s") * 2 + lax.axis_index("c")
    for c in range(RPW // CH):
        base = wid * RPW + c * CH
        pltpu.sync_copy(srcg.at[pl.ds(base, CH)], idx_v)
        pltpu.async_copy(tab.at[idx_v], buf, sem).wait()
        pltpu.sync_copy(buf, out.at[pl.ds(base, CH)])


def kernel(k, v, m_k, m_v, m_u):
    s = jax.nn.softmax(jnp.einsum('bhd,bmd->bhm', k, m_k), axis=-1)
    max_s_hw = jnp.max(s, axis=-1)

    r2, ru, wv, pw = _ranks(max_s_hw, m_u)
    src = _compose(r2, ru, wv, pw)

    vpad = jnp.pad(v, ((0, 0), (0, 0), (0, VW - VDIM)))
    mvpad = jnp.pad(m_v, ((0, 0), (0, 0), (0, VW - VDIM)))
    tab = jnp.concatenate([
        jnp.concatenate([k.reshape(B * M, KDIM), vpad.reshape(B * M, VW)],
                        axis=1),
        jnp.concatenate([m_k.reshape(B * M, KDIM), mvpad.reshape(B * M, VW)],
                        axis=1),
    ], axis=0)

    out = _sc_gather(tab, src.reshape(B * M))
    m_k_new = out[:, :KDIM].reshape(B, M, KDIM)
    m_v_new = out[:, KDIM:KDIM + VDIM].reshape(B, M, VDIM)
    return (m_k_new, m_v_new)


# speedup vs baseline: 1.0343x; 1.0343x over previous
import functools

import jax
import jax.numpy as jnp
from jax import lax
from jax.experimental import pallas as pl
from jax.experimental.pallas import tpu as pltpu
from jax.experimental.pallas import tpu_sc as plsc

B = 4
M = 4096
HW = 4096
KDIM = 256
VDIM = 3
THRESHOLD = 0.35 * 100 / M

TI = 512
NT = M // TI
NW = 32
RPW = B * M // NW
CH = 128
VW = 128
TW = KDIM + VW


def _key(x):
    return 2 * (lax.bitcast_convert_type(x, jnp.int32) - jnp.int32(536870912))


def _rank_body(sc_col_ref, sc_row_ref, mu_col_ref, mu_row_ref,
               r2_ref, ru_ref, wv_ref, pw_ref):
    it = pl.program_id(1)
    si = sc_col_ref[0]
    sj = sc_row_ref[0]
    ki = _key(si)
    kj = _key(sj)
    vi = _key(mu_col_ref[0])
    vj = _key(mu_row_ref[0])
    ii = lax.broadcasted_iota(jnp.int32, (TI, M), 0) + it * TI
    jj = lax.broadcasted_iota(jnp.int32, (TI, M), 1)
    jlt = (jj < ii).astype(jnp.int32)
    r2 = jnp.sum(((kj + jlt) > ki).astype(jnp.float32),
                 axis=1, keepdims=True)
    r2_ref[0] = r2.astype(jnp.int32)
    ru = jnp.sum((vj < (vi + jlt)).astype(jnp.float32),
                 axis=1, keepdims=True)
    ru_ref[0] = ru.astype(jnp.int32)
    wv_ref[0] = (si < THRESHOLD).astype(jnp.int32)
    pw = jnp.sum(((sj < THRESHOLD) & (jlt == 1)).astype(jnp.float32),
                 axis=1, keepdims=True)
    pw_ref[0] = pw.astype(jnp.int32)


def _ranks(scores, m_u):
    sc_col = scores.reshape(B * NT, TI, 1)
    sc_row = scores.reshape(B, 1, M)
    mu_col = m_u.reshape(B * NT, TI, 1)
    mu_row = m_u.reshape(B, 1, M)
    col_spec = pl.BlockSpec((1, TI, 1), lambda b, i: (b * NT + i, 0, 0))
    row_spec = pl.BlockSpec((1, 1, M), lambda b, i: (b, 0, 0))
    out_spec = pl.BlockSpec((1, TI, 1), lambda b, i: (b * NT + i, 0, 0))
    out_sds = jax.ShapeDtypeStruct((B * NT, TI, 1), jnp.int32)
    r2, ru, wv, pw = pl.pallas_call(
        _rank_body,
        grid=(B, NT),
        in_specs=[col_spec, row_spec, col_spec, row_spec],
        out_specs=[out_spec, out_spec, out_spec, out_spec],
        out_shape=[out_sds, out_sds, out_sds, out_sds],
    )(sc_col, sc_row, mu_col, mu_row)
    return (r2.reshape(B, M), ru.reshape(B, M), wv.reshape(B, M),
            pw.reshape(B, M))


def _compose_body(r2_ref, ru_ref, wv_ref, pw_ref, src_ref):
    b = pl.program_id(0)
    it = pl.program_id(1)
    r2r = r2_ref[0]
    rur = ru_ref[0]
    wvr = wv_ref[0]
    pwr = pw_ref[0]
    jcol = lax.broadcasted_iota(jnp.int32, (TI, M), 0) + it * TI
    irow = lax.broadcasted_iota(jnp.int32, (TI, M), 1)
    irow_f = irow.astype(jnp.float32)
    zero = jnp.zeros((), jnp.float32)
    tmat = (pwr == jcol) & (wvr == 1)
    t = jnp.sum(jnp.where(tmat, irow_f, zero), axis=1,
                keepdims=True).astype(jnp.int32)
    srcw = jnp.sum(jnp.where(r2r == t, irow_f, zero), axis=1,
                   keepdims=True).astype(jnp.int32)
    srcd = jnp.sum(jnp.where(rur == jcol, irow_f, zero), axis=1,
                   keepdims=True).astype(jnp.int32)
    cnt = jnp.sum(wvr.astype(jnp.float32)).astype(jnp.int32)
    jvec = jcol[:, :1]
    src_ref[0] = jnp.where(jvec < cnt, srcw + b * M,
                           srcd + b * M + B * M)


def _compose(r2, ru, wv, pw):
    rows = [x.reshape(B, 1, M) for x in (r2, ru, wv, pw)]
    row_spec = pl.BlockSpec((1, 1, M), lambda b, i: (b, 0, 0))
    out_spec = pl.BlockSpec((1, TI, 1), lambda b, i: (b * NT + i, 0, 0))
    src = pl.pallas_call(
        _compose_body,
        grid=(B, NT),
        in_specs=[row_spec] * 4,
        out_specs=out_spec,
        out_shape=jax.ShapeDtypeStruct((B * NT, TI, 1), jnp.int32),
    )(*rows)
    return src.reshape(B, M)


_MESH = plsc.VectorSubcoreMesh(core_axis_name="c", subcore_axis_name="s")


@functools.partial(
    pl.kernel,
    out_type=jax.ShapeDtypeStruct((B * M, TW), jnp.float32),
    mesh=_MESH,
    scratch_types=[
        pltpu.VMEM((CH,), jnp.int32),
        pltpu.VMEM((CH, TW), jnp.float32),
        pltpu.SemaphoreType.DMA,
    ],
)
def _sc_gather(tab, srcg, out, idx_v, buf, sem):
    wid = lax.axis_index("s") * 2 + lax.axis_index("c")
    for c in range(RPW // CH):
        base = wid * RPW + c * CH
        pltpu.sync_copy(srcg.at[pl.ds(base, CH)], idx_v)
        pltpu.async_copy(tab.at[idx_v], buf, sem).wait()
        pltpu.sync_copy(buf, out.at[pl.ds(base, CH)])


def kernel(k, v, m_k, m_v, m_u):
    s = jax.nn.softmax(jnp.einsum('bhd,bmd->bhm', k, m_k), axis=-1)
    max_s_hw = jnp.max(s, axis=-1)

    r2, ru, wv, pw = _ranks(max_s_hw, m_u)
    src = _compose(r2, ru, wv, pw)

    vpad = jnp.pad(v, ((0, 0), (0, 0), (0, VW - VDIM)))
    mvpad = jnp.pad(m_v, ((0, 0), (0, 0), (0, VW - VDIM)))
    tab = jnp.concatenate([
        jnp.concatenate([k.reshape(B * M, KDIM), vpad.reshape(B * M, VW)],
                        axis=1),
        jnp.concatenate([m_k.reshape(B * M, KDIM), mvpad.reshape(B * M, VW)],
                        axis=1),
    ], axis=0)

    out = _sc_gather(tab, src.reshape(B * M))
    m_k_new = out[:, :KDIM].reshape(B, M, KDIM)
    m_v_new = out[:, KDIM:KDIM + VDIM].reshape(B, M, VDIM)
    return (m_k_new, m_v_new)
